# baseline (speedup 1.0000x reference)
import jax
import jax.numpy as jnp
from jax.experimental import pallas as pl


def _mask_overwrite_kernel(out_ref):
    mask = jnp.ones((1,), dtype=jnp.bool_)
    ones = jnp.ones((1,), dtype=jnp.float32)
    zeros = jnp.zeros((1,), dtype=jnp.float32)
    out_ref[...] = jnp.where(mask, ones, zeros)


def kernel(x):
    return pl.pallas_call(
        _mask_overwrite_kernel,
        out_shape=jax.ShapeDtypeStruct((1,), jnp.float32),
    )()
